# Initial kernel scaffold; baseline (speedup 1.0000x reference)
import jax
import jax.numpy as jnp
from jax.experimental import pallas as pl


def kernel(input_ids, labels, W):
    raise NotImplementedError("write your pallas kernel here")


# speedup vs baseline: 1.3476x; 1.3476x over previous
import functools
import math

import jax
import jax.numpy as jnp
from jax import lax
from jax.experimental import pallas as pl
from jax.experimental.pallas import tpu as pltpu
from jax.experimental.pallas import tpu_sc as plsc

_VOCAB = 151936
_D = 1024
_B = 2
_S = 4096
_H = 16
_HEAD = _D // _H
_THETA = 1000000.0

_N = _B * _S
_NC = 2
_NS = 16
_NW = _NC * _NS
_PER_W = _N // _NW
_CHUNK = 64
_NCH = _PER_W // _CHUNK


def _gather_body(ids_hbm, table_hbm, out_hbm, idx_v, rows_v, sem):
    wid = lax.axis_index("s") * _NC + lax.axis_index("c")
    base = wid * _PER_W
    pltpu.sync_copy(ids_hbm.at[wid], idx_v)
    for c in range(_NCH):
        pltpu.async_copy(table_hbm.at[idx_v.at[c]], rows_v, sem).wait()
        pltpu.sync_copy(rows_v, out_hbm.at[pl.ds(base + c * _CHUNK, _CHUNK)])


def _rope_body(inv_ref, cos_ref, sin_ref):
    pos = lax.broadcasted_iota(jnp.int32, (_S, _HEAD), 0).astype(jnp.float32)
    ang = pos * inv_ref[...]
    cos_ref[...] = jnp.cos(ang)
    sin_ref[...] = jnp.sin(ang)


def kernel(input_ids, labels, W):
    ids3 = input_ids.reshape(_NW, _NCH, _CHUNK)

    @functools.partial(
        pl.kernel,
        out_type=jax.ShapeDtypeStruct((_N, _D), jnp.float32),
        mesh=plsc.VectorSubcoreMesh(core_axis_name="c", subcore_axis_name="s"),
        scratch_types=[
            pltpu.VMEM((_NCH, _CHUNK), jnp.int32),
            pltpu.VMEM((_CHUNK, _D), jnp.float32),
            pltpu.SemaphoreType.DMA,
        ],
    )
    def gather_sc(ids_hbm, table_hbm, out_hbm, idx_v, rows_v, sem):
        _gather_body(ids_hbm, table_hbm, out_hbm, idx_v, rows_v, sem)

    flat = gather_sc(ids3, W)
    hidden_states = flat.reshape(_B, _S, _D)

    half = jnp.arange(0, _HEAD, 2, dtype=jnp.float32) / _HEAD
    inv_freq = 1.0 / (_THETA ** half)
    inv_full = jnp.concatenate([inv_freq, inv_freq])[None, :]

    cos2, sin2 = pl.pallas_call(
        _rope_body,
        out_shape=[
            jax.ShapeDtypeStruct((_S, _HEAD), jnp.float32),
            jax.ShapeDtypeStruct((_S, _HEAD), jnp.float32),
        ],
    )(inv_full)
    cos = cos2[None]
    sin = sin2[None]

    requires_grad_idx = jnp.array([3], dtype=jnp.int32)
    cache_position = jnp.arange(0, _S, dtype=jnp.int32)
    position_ids = cache_position[None, :]
    return (requires_grad_idx, cos, sin, hidden_states, position_ids,
            cache_position, labels)


# speedup vs baseline: 1.3802x; 1.0242x over previous
import functools
import math

import jax
import jax.numpy as jnp
from jax import lax
from jax.experimental import pallas as pl
from jax.experimental.pallas import tpu as pltpu
from jax.experimental.pallas import tpu_sc as plsc

_VOCAB = 151936
_D = 1024
_B = 2
_S = 4096
_H = 16
_HEAD = _D // _H
_THETA = 1000000.0

_N = _B * _S
_NC = 2
_NS = 16
_NW = _NC * _NS
_PER_W = _N // _NW
_CHUNK = 32
_NCH = _PER_W // _CHUNK


def _gather_body(ids_hbm, table_hbm, out_hbm, idx_v, rows_v, gsem, wsem):
    wid = lax.axis_index("s") * _NC + lax.axis_index("c")
    base = wid * _PER_W
    pltpu.sync_copy(ids_hbm.at[wid], idx_v)

    def gather(c):
        return pltpu.async_copy(table_hbm.at[idx_v.at[c]], rows_v.at[c % 2],
                                gsem)

    def write(c):
        return pltpu.async_copy(rows_v.at[c % 2],
                                out_hbm.at[pl.ds(base + c * _CHUNK, _CHUNK)],
                                wsem)

    gathers = [gather(0)]
    writes = []
    for c in range(_NCH):
        gathers[c].wait()
        if c + 1 < _NCH:
            if c >= 1:
                writes[c - 1].wait()
            gathers.append(gather(c + 1))
        writes.append(write(c))
    writes[_NCH - 2].wait()
    writes[_NCH - 1].wait()


def _rope_body(inv_ref, cos_ref, sin_ref):
    pos = lax.broadcasted_iota(jnp.int32, (_S, _HEAD), 0).astype(jnp.float32)
    ang = pos * inv_ref[...]
    cos_ref[...] = jnp.cos(ang)
    sin_ref[...] = jnp.sin(ang)


def kernel(input_ids, labels, W):
    ids3 = input_ids.reshape(_NW, _NCH, _CHUNK)

    @functools.partial(
        pl.kernel,
        out_type=jax.ShapeDtypeStruct((_N, _D), jnp.float32),
        mesh=plsc.VectorSubcoreMesh(core_axis_name="c", subcore_axis_name="s"),
        scratch_types=[
            pltpu.VMEM((_NCH, _CHUNK), jnp.int32),
            pltpu.VMEM((2, _CHUNK, _D), jnp.float32),
            pltpu.SemaphoreType.DMA,
            pltpu.SemaphoreType.DMA,
        ],
    )
    def gather_sc(ids_hbm, table_hbm, out_hbm, idx_v, rows_v, gsem, wsem):
        _gather_body(ids_hbm, table_hbm, out_hbm, idx_v, rows_v, gsem, wsem)

    flat = gather_sc(ids3, W)
    hidden_states = flat.reshape(_B, _S, _D)

    half = jnp.arange(0, _HEAD, 2, dtype=jnp.float32) / _HEAD
    inv_freq = 1.0 / (_THETA ** half)
    inv_full = jnp.concatenate([inv_freq, inv_freq])[None, :]

    cos2, sin2 = pl.pallas_call(
        _rope_body,
        out_shape=[
            jax.ShapeDtypeStruct((_S, _HEAD), jnp.float32),
            jax.ShapeDtypeStruct((_S, _HEAD), jnp.float32),
        ],
    )(inv_full)
    cos = cos2[None]
    sin = sin2[None]

    requires_grad_idx = jnp.array([3], dtype=jnp.int32)
    cache_position = jnp.arange(0, _S, dtype=jnp.int32)
    position_ids = cache_position[None, :]
    return (requires_grad_idx, cos, sin, hidden_states, position_ids,
            cache_position, labels)


# speedup vs baseline: 1.4350x; 1.0397x over previous
import functools
import math

import jax
import jax.numpy as jnp
from jax import lax
from jax.experimental import pallas as pl
from jax.experimental.pallas import tpu as pltpu
from jax.experimental.pallas import tpu_sc as plsc

_VOCAB = 151936
_D = 1024
_B = 2
_S = 4096
_H = 16
_HEAD = _D // _H
_THETA = 1000000.0

_N = _B * _S
_NC = 2
_NS = 16
_NW = _NC * _NS
_PER_W = _N // _NW
_CHUNK = 32
_NCH = _PER_W // _CHUNK
_NBUF = 3


def _gather_body(ids_hbm, table_hbm, out_hbm, idx_v, rows_v, gsem, wsem):
    wid = lax.axis_index("s") * _NC + lax.axis_index("c")
    base = wid * _PER_W
    pltpu.sync_copy(ids_hbm.at[wid], idx_v)

    def gather(c):
        return pltpu.async_copy(table_hbm.at[idx_v.at[c]],
                                rows_v.at[c % _NBUF], gsem)

    def write(c):
        return pltpu.async_copy(rows_v.at[c % _NBUF],
                                out_hbm.at[pl.ds(base + c * _CHUNK, _CHUNK)],
                                wsem)

    gathers = [gather(0), gather(1)]
    writes = []
    for c in range(_NCH):
        gathers[c].wait()
        if c + 2 < _NCH:
            if c >= 1:
                writes[c - 1].wait()
            gathers.append(gather(c + 2))
        writes.append(write(c))
    for c in range(max(0, _NCH - 3), _NCH):
        writes[c].wait()


def _rope_body(inv_ref, cos_ref, sin_ref):
    pos = lax.broadcasted_iota(jnp.int32, (_S, _HEAD), 0).astype(jnp.float32)
    ang = pos * inv_ref[...]
    cos_ref[...] = jnp.cos(ang)
    sin_ref[...] = jnp.sin(ang)


def kernel(input_ids, labels, W):
    ids3 = input_ids.reshape(_NW, _NCH, _CHUNK)

    @functools.partial(
        pl.kernel,
        out_type=jax.ShapeDtypeStruct((_N, _D), jnp.float32),
        mesh=plsc.VectorSubcoreMesh(core_axis_name="c", subcore_axis_name="s"),
        scratch_types=[
            pltpu.VMEM((_NCH, _CHUNK), jnp.int32),
            pltpu.VMEM((_NBUF, _CHUNK, _D), jnp.float32),
            pltpu.SemaphoreType.DMA,
            pltpu.SemaphoreType.DMA,
        ],
    )
    def gather_sc(ids_hbm, table_hbm, out_hbm, idx_v, rows_v, gsem, wsem):
        _gather_body(ids_hbm, table_hbm, out_hbm, idx_v, rows_v, gsem, wsem)

    flat = gather_sc(ids3, W)
    hidden_states = flat.reshape(_B, _S, _D)

    half = jnp.arange(0, _HEAD, 2, dtype=jnp.float32) / _HEAD
    inv_freq = 1.0 / (_THETA ** half)
    inv_full = jnp.concatenate([inv_freq, inv_freq])[None, :]

    cos2, sin2 = pl.pallas_call(
        _rope_body,
        out_shape=[
            jax.ShapeDtypeStruct((_S, _HEAD), jnp.float32),
            jax.ShapeDtypeStruct((_S, _HEAD), jnp.float32),
        ],
    )(inv_full)
    cos = cos2[None]
    sin = sin2[None]

    requires_grad_idx = jnp.array([3], dtype=jnp.int32)
    cache_position = jnp.arange(0, _S, dtype=jnp.int32)
    position_ids = cache_position[None, :]
    return (requires_grad_idx, cos, sin, hidden_states, position_ids,
            cache_position, labels)


# speedup vs baseline: 1.4495x; 1.0101x over previous
import functools
import math

import jax
import jax.numpy as jnp
from jax import lax
from jax.experimental import pallas as pl
from jax.experimental.pallas import tpu as pltpu
from jax.experimental.pallas import tpu_sc as plsc

_VOCAB = 151936
_D = 1024
_B = 2
_S = 4096
_H = 16
_HEAD = _D // _H
_THETA = 1000000.0

_N = _B * _S
_NC = 2
_NS = 16
_NW = _NC * _NS
_PER_W = _N // _NW
_CHUNK = 32
_NCH = _PER_W // _CHUNK
_NBUF = 3


def _gather_body(ids_hbm, table_hbm, out_hbm, idx_v, rows_v, gsem, wsem):
    wid = lax.axis_index("s") * _NC + lax.axis_index("c")
    wpb = _S // _PER_W
    b = wid // wpb
    s0 = (wid % wpb) * _PER_W
    pltpu.sync_copy(ids_hbm.at[b, pl.ds(s0, _PER_W)], idx_v)

    def gather(c):
        return pltpu.async_copy(table_hbm.at[idx_v.at[pl.ds(c * _CHUNK, _CHUNK)]],
                                rows_v.at[c % _NBUF], gsem)

    def write(c):
        return pltpu.async_copy(rows_v.at[c % _NBUF],
                                out_hbm.at[b, pl.ds(s0 + c * _CHUNK, _CHUNK)],
                                wsem)

    gathers = [gather(0), gather(1)]
    writes = []
    for c in range(_NCH):
        gathers[c].wait()
        if c + 2 < _NCH:
            if c >= 1:
                writes[c - 1].wait()
            gathers.append(gather(c + 2))
        writes.append(write(c))
    for c in range(max(0, _NCH - 3), _NCH):
        writes[c].wait()


def _rope_body(inv_ref, cos_ref, sin_ref):
    pos = lax.broadcasted_iota(jnp.int32, (_S, _HEAD), 0).astype(jnp.float32)
    ang = pos * inv_ref[...]
    cos_ref[...] = jnp.cos(ang)
    sin_ref[...] = jnp.sin(ang)


def kernel(input_ids, labels, W):
    @functools.partial(
        pl.kernel,
        out_type=jax.ShapeDtypeStruct((_B, _S, _D), jnp.float32),
        mesh=plsc.VectorSubcoreMesh(core_axis_name="c", subcore_axis_name="s"),
        scratch_types=[
            pltpu.VMEM((_PER_W,), jnp.int32),
            pltpu.VMEM((_NBUF, _CHUNK, _D), jnp.float32),
            pltpu.SemaphoreType.DMA,
            pltpu.SemaphoreType.DMA,
        ],
    )
    def gather_sc(ids_hbm, table_hbm, out_hbm, idx_v, rows_v, gsem, wsem):
        _gather_body(ids_hbm, table_hbm, out_hbm, idx_v, rows_v, gsem, wsem)

    hidden_states = gather_sc(input_ids, W)

    half = jnp.arange(0, _HEAD, 2, dtype=jnp.float32) / _HEAD
    inv_freq = 1.0 / (_THETA ** half)
    inv_full = jnp.concatenate([inv_freq, inv_freq])[None, :]

    cos2, sin2 = pl.pallas_call(
        _rope_body,
        out_shape=[
            jax.ShapeDtypeStruct((_S, _HEAD), jnp.float32),
            jax.ShapeDtypeStruct((_S, _HEAD), jnp.float32),
        ],
    )(inv_full)
    cos = cos2[None]
    sin = sin2[None]

    requires_grad_idx = jnp.array([3], dtype=jnp.int32)
    cache_position = jnp.arange(0, _S, dtype=jnp.int32)
    position_ids = cache_position[None, :]
    return (requires_grad_idx, cos, sin, hidden_states, position_ids,
            cache_position, labels)
